# Initial kernel scaffold; baseline (speedup 1.0000x reference)
import jax
import jax.numpy as jnp
from jax.experimental import pallas as pl


def kernel(features, labels, running_mean, running_var, num_samples_tracked):
    raise NotImplementedError("write your pallas kernel here")


# speedup vs baseline: 2.1490x; 2.1490x over previous
import functools

import jax
import jax.numpy as jnp
from jax import lax
from jax.experimental import pallas as pl
from jax.experimental.pallas import tpu as pltpu
from jax.experimental.pallas import tpu_sc as plsc

NC = 2
NS = 16
NW = NC * NS
N = 65536
D = 512
NB = 50
NBP = 64
ROWS_PER_W = N // NW
C = 64
G = ROWS_PER_W // C
MOM = 0.9

_mesh = plsc.VectorSubcoreMesh(core_axis_name="c", subcore_axis_name="s",
                               num_cores=NC, num_subcores=NS)


def _sc_body(feat, lbl, zeros, psum, psq, pcnt, featbuf, lblbuf, acc_s, acc_q,
             acc_c):
    c = lax.axis_index("c")
    s = lax.axis_index("s")
    wid = s * NC + c

    pltpu.sync_copy(zeros, acc_s)
    pltpu.sync_copy(zeros, acc_q)
    zv = jnp.zeros((16,), jnp.float32)
    for r in range(NBP):
        acc_c[r, :] = zv
    lane = jnp.arange(16, dtype=jnp.int32)
    marker = jnp.where(lane == 0, 1.0, 0.0).astype(jnp.float32)

    base0 = wid * ROWS_PER_W

    def chunk(g, carry):
        base = base0 + g * C
        pltpu.sync_copy(feat.at[pl.ds(base, C)], featbuf)
        pltpu.sync_copy(lbl.at[pl.ds(base, C)], lblbuf)

        def group(gr, carry2):
            lv = lblbuf[pl.ds(gr * 16, 16)]
            for l in range(16):
                lb = lv[l]
                r = gr * 16 + l

                def blk(j, _, r=r, lb=lb):
                    v = featbuf[r, pl.ds(j * 16, 16)]
                    plsc.addupdate(acc_s.at[lb, pl.ds(j * 16, 16)], v)
                    plsc.addupdate(acc_q.at[lb, pl.ds(j * 16, 16)], v * v)
                    return 0

                lax.fori_loop(0, D // 16, blk, 0, unroll=8)
                plsc.addupdate(acc_c.at[lb], marker)
            return carry2

        lax.fori_loop(0, C // 16, group, 0)
        return carry

    lax.fori_loop(0, G, chunk, 0)

    pltpu.sync_copy(acc_s, psum.at[wid])
    pltpu.sync_copy(acc_q, psq.at[wid])
    pltpu.sync_copy(acc_c, pcnt.at[wid])


_sc_call = functools.partial(
    pl.kernel,
    out_type=(
        jax.ShapeDtypeStruct((NW, NBP, D), jnp.float32),
        jax.ShapeDtypeStruct((NW, NBP, D), jnp.float32),
        jax.ShapeDtypeStruct((NW, NBP, 16), jnp.float32),
    ),
    mesh=_mesh,
    scratch_types=[
        pltpu.VMEM((C, D), jnp.float32),
        pltpu.VMEM((C,), jnp.int32),
        pltpu.VMEM((NBP, D), jnp.float32),
        pltpu.VMEM((NBP, D), jnp.float32),
        pltpu.VMEM((NBP, 16), jnp.float32),
    ],
)(_sc_body)


def _fin_body(ps, pq, pc, rm, rv, nst, om, ov, on):
    sx = jnp.sum(ps[...], axis=0)[:NB]
    qx = jnp.sum(pq[...], axis=0)[:NB]
    cnt = jnp.sum(pc[...], axis=0)[:NB, 0:1]
    safe_n = jnp.maximum(cnt, 1.0)
    mean = sx / safe_n
    denom = jnp.maximum(cnt - 1.0, 1.0)
    var_u = (qx - cnt * mean * mean) / denom
    var_b = qx / safe_n - mean * mean
    var = jnp.where(cnt > 1.0, var_u, var_b)
    present = cnt > 0.0
    om[...] = jnp.where(present, (1.0 - MOM) * mean + MOM * rm[...], rm[...])
    ov[...] = jnp.where(present, (1.0 - MOM) * var + MOM * rv[...], rv[...])
    on[...] = nst[...] + cnt


_fin_call = pl.pallas_call(
    _fin_body,
    out_shape=(
        jax.ShapeDtypeStruct((NB, D), jnp.float32),
        jax.ShapeDtypeStruct((NB, D), jnp.float32),
        jax.ShapeDtypeStruct((NB, 1), jnp.float32),
    ),
)


def kernel(features, labels, running_mean, running_var, num_samples_tracked):
    zeros = jnp.zeros((NBP, D), jnp.float32)
    psum, psq, pcnt = _sc_call(features, labels, zeros)
    new_mean, new_var, new_num = _fin_call(
        psum, psq, pcnt, running_mean, running_var,
        num_samples_tracked.reshape(NB, 1))
    return new_mean, new_var, new_num.reshape(NB)


# speedup vs baseline: 2.5097x; 1.1678x over previous
import functools

import jax
import jax.numpy as jnp
from jax import lax
from jax.experimental import pallas as pl
from jax.experimental.pallas import tpu as pltpu
from jax.experimental.pallas import tpu_sc as plsc

NC = 2
NS = 16
NW = NC * NS
N = 65536
D = 512
NB = 50
ROWS_PER_W = N // NW
C = 32
G = ROWS_PER_W // C
MOM = 0.9

_mesh = plsc.VectorSubcoreMesh(core_axis_name="c", subcore_axis_name="s",
                               num_cores=NC, num_subcores=NS)


def _sc_body(feat, lbl, zeros, psum, psq, pcnt, featbuf, lblbuf, acc_s, acc_q,
             acc_c, fsem):
    c = lax.axis_index("c")
    s = lax.axis_index("s")
    wid = s * NC + c
    base0 = wid * ROWS_PER_W

    pltpu.sync_copy(zeros, acc_s)
    pltpu.sync_copy(zeros, acc_q)
    zv = jnp.zeros((16,), jnp.float32)
    for r in range(NB):
        acc_c[r, :] = zv
    pltpu.sync_copy(lbl.at[pl.ds(base0, ROWS_PER_W)], lblbuf)
    lane = jnp.arange(16, dtype=jnp.int32)
    marker = jnp.where(lane == 0, 1.0, 0.0).astype(jnp.float32)

    def fetch(g):
        slot = lax.rem(g, 2)
        return pltpu.async_copy(
            feat.at[pl.ds(base0 + g * C, C)],
            featbuf.at[pl.ds(slot * C, C)],
            fsem.at[slot],
        )

    fetch(0)

    def chunk(g, carry):
        slot = lax.rem(g, 2)
        pltpu.make_async_copy(
            feat.at[pl.ds(base0 + g * C, C)],
            featbuf.at[pl.ds(slot * C, C)],
            fsem.at[slot],
        ).wait()

        @pl.when(g + 1 < G)
        def _():
            fetch(g + 1)

        row0 = slot * C

        def group(gr, carry2):
            lv = lblbuf[pl.ds(g * C + gr * 16, 16)]
            for l in range(16):
                lb = lv[l]
                r = row0 + gr * 16 + l

                def blk(j, _, r=r, lb=lb):
                    v = featbuf[r, pl.ds(j * 16, 16)]
                    plsc.addupdate(acc_s.at[lb, pl.ds(j * 16, 16)], v)
                    plsc.addupdate(acc_q.at[lb, pl.ds(j * 16, 16)], v * v)
                    return 0

                lax.fori_loop(0, D // 16, blk, 0, unroll=8)
                plsc.addupdate(acc_c.at[lb], marker)
            return carry2

        lax.fori_loop(0, C // 16, group, 0)
        return carry

    lax.fori_loop(0, G, chunk, 0)

    pltpu.sync_copy(acc_s, psum.at[wid])
    pltpu.sync_copy(acc_q, psq.at[wid])
    pltpu.sync_copy(acc_c, pcnt.at[wid])


_sc_call = functools.partial(
    pl.kernel,
    out_type=(
        jax.ShapeDtypeStruct((NW, NB, D), jnp.float32),
        jax.ShapeDtypeStruct((NW, NB, D), jnp.float32),
        jax.ShapeDtypeStruct((NW, NB, 16), jnp.float32),
    ),
    mesh=_mesh,
    scratch_types=[
        pltpu.VMEM((2 * C, D), jnp.float32),
        pltpu.VMEM((ROWS_PER_W,), jnp.int32),
        pltpu.VMEM((NB, D), jnp.float32),
        pltpu.VMEM((NB, D), jnp.float32),
        pltpu.VMEM((NB, 16), jnp.float32),
        pltpu.SemaphoreType.DMA((2,)),
    ],
)(_sc_body)


def _fin_body(ps, pq, pc, rm, rv, nst, om, ov, on):
    sx = jnp.sum(ps[...], axis=0)
    qx = jnp.sum(pq[...], axis=0)
    cnt = jnp.sum(pc[...], axis=0)[:, 0:1]
    safe_n = jnp.maximum(cnt, 1.0)
    mean = sx / safe_n
    denom = jnp.maximum(cnt - 1.0, 1.0)
    var_u = (qx - cnt * mean * mean) / denom
    var_b = qx / safe_n - mean * mean
    var = jnp.where(cnt > 1.0, var_u, var_b)
    present = cnt > 0.0
    om[...] = jnp.where(present, (1.0 - MOM) * mean + MOM * rm[...], rm[...])
    ov[...] = jnp.where(present, (1.0 - MOM) * var + MOM * rv[...], rv[...])
    on[...] = nst[...] + cnt


_fin_call = pl.pallas_call(
    _fin_body,
    out_shape=(
        jax.ShapeDtypeStruct((NB, D), jnp.float32),
        jax.ShapeDtypeStruct((NB, D), jnp.float32),
        jax.ShapeDtypeStruct((NB, 1), jnp.float32),
    ),
)


def kernel(features, labels, running_mean, running_var, num_samples_tracked):
    zeros = jnp.zeros((NB, D), jnp.float32)
    psum, psq, pcnt = _sc_call(features, labels, zeros)
    new_mean, new_var, new_num = _fin_call(
        psum, psq, pcnt, running_mean, running_var,
        num_samples_tracked.reshape(NB, 1))
    return new_mean, new_var, new_num.reshape(NB)
